# baseline (speedup 1.0000x reference)
import jax
import jax.numpy as jnp
from jax import lax
from jax.experimental import pallas as pl
from jax.experimental.pallas import tpu as pltpu, tpu_sc as plsc

B = 16384
D = 16
NV = 1000000
P = 1000064
NC, NS = 2, 16
NW = NC * NS

PW = 3584
NPF = NV // PW
TAIL = NV - NPF * PW
NFULL = 4 * NPF
NI = (NFULL + NW - 1) // NW

BPW = B // NW
ICH = 128
NJ = BPW // ICH
NCH = BPW // 16


def _detile_body(wot, wit, tails, fo, fi, stg0, stg1, stg2, outb0, outb1,
                 tstg, si0, si1, si2, so0, so1):
    wid = lax.axis_index("s") * NC + lax.axis_index("c")
    stgs, outbs = [stg0, stg1, stg2], [outb0, outb1]
    sins, souts = [si0, si1, si2], [so0, so1]

    def decode(u):
        return u % 2, (u // 2) % 2, u // 4

    def fire_in(u, par):
        tbl, g, piece = decode(u)
        c0 = piece * PW

        @pl.when(tbl == 0)
        def _():
            pltpu.async_copy(wot.at[pl.ds(g * 8, 8), pl.ds(c0, PW)],
                             stgs[par], sins[par])

        @pl.when(tbl == 1)
        def _():
            pltpu.async_copy(wit.at[pl.ds(g * 8, 8), pl.ds(c0, PW)],
                             stgs[par], sins[par])

    def wait_in(par):
        pltpu.make_async_copy(wot.at[pl.ds(0, 8), pl.ds(0, PW)],
                              stgs[par], sins[par]).wait()

    def extract(par, opar):
        s, o = stgs[par], outbs[opar]

        def row128(c128, carry):
            for h in range(8):
                sl = pl.ds(c128 * 128 + h * 16, 16)
                for p in range(4):
                    pk = plsc.pack(s[2 * p, sl], s[2 * p + 1, sl],
                                   format=plsc.PackFormat.INTERLEAVED)
                    o[p, sl] = plsc.bitcast(pk, jnp.int32)
            return carry

        lax.fori_loop(0, PW // 128, row128, 0)

    def fire_out(u, par):
        tbl, g, piece = decode(u)
        c0 = piece * PW
        for p in range(4):
            dst_off = (g * 4 + p) * P + c0

            @pl.when(tbl == 0)
            def _():
                pltpu.async_copy(outbs[par].at[p, pl.ds(0, PW)],
                                 fo.at[pl.ds(dst_off, PW)], souts[par])

            @pl.when(tbl == 1)
            def _():
                pltpu.async_copy(outbs[par].at[p, pl.ds(0, PW)],
                                 fi.at[pl.ds(dst_off, PW)], souts[par])

    def wait_out(par):
        for p in range(4):
            pltpu.make_async_copy(outbs[par].at[p, pl.ds(0, PW)],
                                  fo.at[pl.ds(0, PW)], souts[par]).wait()

    @pl.when(wid < NFULL)
    def _():
        fire_in(wid, 0)

    @pl.when(wid + NW < NFULL)
    def _():
        fire_in(wid + NW, 1)

    NI6 = (NI + 5) // 6

    def body(i6, carry):
        for par6 in range(6):
            i = 6 * i6 + par6
            spar = par6 % 3
            opar = par6 % 2
            u = wid + NW * i
            un2 = wid + NW * (i + 2)

            @pl.when(un2 < NFULL)
            def _():
                fire_in(un2, (par6 + 2) % 3)

            @pl.when((i >= 2) & (wid + NW * (i - 2) < NFULL))
            def _():
                wait_out(opar)

            @pl.when(u < NFULL)
            def _():
                wait_in(spar)
                extract(spar, opar)
                fire_out(u, opar)

        return carry

    lax.fori_loop(0, NI6, body, 0)
    for j in (6 * NI6 - 2, 6 * NI6 - 1):
        @pl.when(wid + NW * j < NFULL)
        def _():
            wait_out(j % 2)

    def do_tail(tbl_idx, dst):
        pltpu.async_copy(tails.at[tbl_idx], tstg, si0).wait()

        def row16(c16, carry):
            sl = pl.ds(c16 * 16, 16)
            for dp in range(8):
                pk = plsc.pack(tstg[2 * dp, sl], tstg[2 * dp + 1, sl],
                               format=plsc.PackFormat.INTERLEAVED)
                outb0[dp % 4, pl.ds((dp // 4) * 128 + c16 * 16, 16)] = (
                    plsc.bitcast(pk, jnp.int32))
            return carry

        lax.fori_loop(0, TAIL // 16, row16, 0)
        cps = []
        for dp in range(8):
            cps.append(pltpu.async_copy(
                outb0.at[dp % 4, pl.ds((dp // 4) * 128, TAIL)],
                dst.at[pl.ds(dp * P + NPF * PW, TAIL)], so0))
        for c in cps:
            c.wait()

    @pl.when(wid == 0)
    def _():
        do_tail(0, fo)

    @pl.when(wid == 1)
    def _():
        do_tail(1, fi)


def _gather_body(uids, iids, fo, wh, fi, out, ids_u, ids_v, idx_u, idx_v,
                 ut, vt, whv, outv, sem_u, sem_v):
    wid = lax.axis_index("s") * NC + lax.axis_index("c")
    base = wid * BPW

    pltpu.sync_copy(uids.at[pl.ds(base, BPW)], ids_u)
    pltpu.sync_copy(iids.at[pl.ds(base, BPW)], ids_v)
    pltpu.sync_copy(wh, whv)

    copies = []
    for dp in range(D // 2):
        doff = jnp.full((16,), dp * P, jnp.int32)
        for t in range(BPW // 16):
            sl = pl.ds(t * 16, 16)
            idx_u[dp, sl] = ids_u[sl] + doff
            idx_v[dp, sl] = ids_v[sl] + doff
        copies.append(pltpu.async_copy(
            fo.at[idx_u.at[dp]], ut.at[dp], sem_u))
        copies.append(pltpu.async_copy(
            fi.at[idx_v.at[dp]], vt.at[dp], sem_v))
    for c in copies:
        c.wait()

    wh_rows = [whv[pl.ds(d * 16, 16)] for d in range(D)]

    def chunk(c, carry):
        s = c * 16
        ucols, vcols = [], []
        for dp in range(D // 2):
            ua, ub = plsc.unpack(
                plsc.bitcast(ut[dp, pl.ds(s, 16)], jnp.bfloat16),
                format=plsc.PackFormat.INTERLEAVED)
            va, vb = plsc.unpack(
                plsc.bitcast(vt[dp, pl.ds(s, 16)], jnp.bfloat16),
                format=plsc.PackFormat.INTERLEAVED)
            ucols += [ua, ub]
            vcols += [va, vb]
        acc = jnp.zeros((16,), jnp.float32)
        for d in range(D):
            t = jnp.zeros((16,), jnp.float32)
            for e in range(D):
                t = t + wh_rows[d][e] * vcols[e]
            acc = acc + ucols[d] * t
        outv[pl.ds(s, 16)] = acc
        return carry

    lax.fori_loop(0, NCH, chunk, 0)
    pltpu.sync_copy(outv, out.at[pl.ds(base, BPW)])


def kernel(user_ids, item_ids, W_o, W_h, W_i):
    mesh = plsc.VectorSubcoreMesh(core_axis_name="c", subcore_axis_name="s")

    detile = pl.kernel(
        _detile_body,
        out_type=(jax.ShapeDtypeStruct((D // 2 * P,), jnp.int32),
                  jax.ShapeDtypeStruct((D // 2 * P,), jnp.int32)),
        mesh=mesh,
        compiler_params=pltpu.CompilerParams(needs_layout_passes=False),
        scratch_types=[
            pltpu.VMEM((8, PW), jnp.float32),
            pltpu.VMEM((8, PW), jnp.float32),
            pltpu.VMEM((8, PW), jnp.float32),
            pltpu.VMEM((4, PW), jnp.int32),
            pltpu.VMEM((4, PW), jnp.int32),
            pltpu.VMEM((D, 128), jnp.float32),
            pltpu.SemaphoreType.DMA,
            pltpu.SemaphoreType.DMA,
            pltpu.SemaphoreType.DMA,
            pltpu.SemaphoreType.DMA,
            pltpu.SemaphoreType.DMA,
        ],
    )
    tails = jnp.stack([
        jnp.pad(W_o[NPF * PW:].T, ((0, 0), (0, 128 - TAIL))),
        jnp.pad(W_i[NPF * PW:].T, ((0, 0), (0, 128 - TAIL))),
    ])
    fo, fi = detile(W_o.T, W_i.T, tails)

    gather = pl.kernel(
        _gather_body,
        out_type=jax.ShapeDtypeStruct((B,), jnp.float32),
        mesh=mesh,
        compiler_params=pltpu.CompilerParams(
            needs_layout_passes=False, use_tc_tiling_on_sc=False),
        scratch_types=[
            pltpu.VMEM((BPW,), jnp.int32),
            pltpu.VMEM((BPW,), jnp.int32),
            pltpu.VMEM((D // 2, BPW), jnp.int32),
            pltpu.VMEM((D // 2, BPW), jnp.int32),
            pltpu.VMEM((D // 2, BPW), jnp.int32),
            pltpu.VMEM((D // 2, BPW), jnp.int32),
            pltpu.VMEM((D * D,), jnp.float32),
            pltpu.VMEM((BPW,), jnp.float32),
            pltpu.SemaphoreType.DMA,
            pltpu.SemaphoreType.DMA,
        ],
    )
    return gather(user_ids, item_ids, fo, W_h.reshape(D * D), fi)


# speedup vs baseline: 1.0243x; 1.0243x over previous
import jax
import jax.numpy as jnp
from jax import lax
from jax.experimental import pallas as pl
from jax.experimental.pallas import tpu as pltpu, tpu_sc as plsc

B = 16384
D = 16
NV = 1000000
P = 1000064
NC, NS = 2, 16
NW = NC * NS

PW = 3584
NPF = NV // PW
TAIL = NV - NPF * PW
NFULL = 4 * NPF
NI = (NFULL + NW - 1) // NW

BPW = B // NW
ICH = 128
NJ = BPW // ICH
NCH = BPW // 16


def _detile_body(wot, wit, tails, fo, fi, stg0, stg1, outb0, outb1, tstg,
                 si0, si1, so0, so1):
    wid = lax.axis_index("s") * NC + lax.axis_index("c")
    stgs, outbs = [stg0, stg1], [outb0, outb1]
    sins, souts = [si0, si1], [so0, so1]

    def decode(u):
        return u % 2, (u // 2) % 2, u // 4

    def fire_in(u, par):
        tbl, g, piece = decode(u)
        c0 = piece * PW

        @pl.when(tbl == 0)
        def _():
            pltpu.async_copy(wot.at[pl.ds(g * 8, 8), pl.ds(c0, PW)],
                             stgs[par], sins[par])

        @pl.when(tbl == 1)
        def _():
            pltpu.async_copy(wit.at[pl.ds(g * 8, 8), pl.ds(c0, PW)],
                             stgs[par], sins[par])

    def wait_in(par):
        pltpu.make_async_copy(wot.at[pl.ds(0, 8), pl.ds(0, PW)],
                              stgs[par], sins[par]).wait()

    def extract(par):
        s, o = stgs[par], outbs[par]

        def row128(c128, carry):
            for h in range(8):
                sl = pl.ds(c128 * 128 + h * 16, 16)
                for p in range(4):
                    pk = plsc.pack(s[2 * p, sl], s[2 * p + 1, sl],
                                   format=plsc.PackFormat.INTERLEAVED)
                    o[p, sl] = plsc.bitcast(pk, jnp.int32)
            return carry

        lax.fori_loop(0, PW // 128, row128, 0)

    def fire_out(u, par):
        tbl, g, piece = decode(u)
        c0 = piece * PW
        for p in range(4):
            dst_off = (g * 4 + p) * P + c0

            @pl.when(tbl == 0)
            def _():
                pltpu.async_copy(outbs[par].at[p, pl.ds(0, PW)],
                                 fo.at[pl.ds(dst_off, PW)], souts[par])

            @pl.when(tbl == 1)
            def _():
                pltpu.async_copy(outbs[par].at[p, pl.ds(0, PW)],
                                 fi.at[pl.ds(dst_off, PW)], souts[par])

    def wait_out(par):
        for p in range(4):
            pltpu.make_async_copy(outbs[par].at[p, pl.ds(0, PW)],
                                  fo.at[pl.ds(0, PW)], souts[par]).wait()

    @pl.when(wid < NFULL)
    def _():
        fire_in(wid, 0)

    NI2 = (NI + 1) // 2

    def body(i2, carry):
        for par in range(2):
            i = 2 * i2 + par
            u = wid + NW * i
            un = wid + NW * (i + 1)

            @pl.when(un < NFULL)
            def _():
                fire_in(un, 1 - par)

            @pl.when((i >= 2) & (wid + NW * (i - 2) < NFULL))
            def _():
                wait_out(par)

            @pl.when(u < NFULL)
            def _():
                wait_in(par)
                extract(par)
                fire_out(u, par)

        return carry

    lax.fori_loop(0, NI2, body, 0)
    for j in (2 * NI2 - 2, 2 * NI2 - 1):
        @pl.when(wid + NW * j < NFULL)
        def _():
            wait_out(j % 2)

    def do_tail(tbl_idx, dst):
        pltpu.async_copy(tails.at[tbl_idx], tstg, si0).wait()

        def row16(c16, carry):
            sl = pl.ds(c16 * 16, 16)
            for dp in range(8):
                pk = plsc.pack(tstg[2 * dp, sl], tstg[2 * dp + 1, sl],
                               format=plsc.PackFormat.INTERLEAVED)
                outb0[dp % 4, pl.ds((dp // 4) * 128 + c16 * 16, 16)] = (
                    plsc.bitcast(pk, jnp.int32))
            return carry

        lax.fori_loop(0, TAIL // 16, row16, 0)
        cps = []
        for dp in range(8):
            cps.append(pltpu.async_copy(
                outb0.at[dp % 4, pl.ds((dp // 4) * 128, TAIL)],
                dst.at[pl.ds(dp * P + NPF * PW, TAIL)], so0))
        for c in cps:
            c.wait()

    @pl.when(wid == 0)
    def _():
        do_tail(0, fo)

    @pl.when(wid == 1)
    def _():
        do_tail(1, fi)


def _gather_body(uids, iids, fo, wh, fi, out, ids_u, ids_v, idx_u, idx_v,
                 ut, vt, whv, outv, sem_u, sem_v):
    wid = lax.axis_index("s") * NC + lax.axis_index("c")
    base = wid * BPW

    pltpu.sync_copy(uids.at[pl.ds(base, BPW)], ids_u)
    pltpu.sync_copy(iids.at[pl.ds(base, BPW)], ids_v)
    pltpu.sync_copy(wh, whv)

    copies = []
    for dp in range(D // 2):
        doff = jnp.full((16,), dp * P, jnp.int32)
        for t in range(BPW // 16):
            sl = pl.ds(t * 16, 16)
            idx_u[dp, sl] = ids_u[sl] + doff
            idx_v[dp, sl] = ids_v[sl] + doff
        copies.append(pltpu.async_copy(
            fo.at[idx_u.at[dp]], ut.at[dp], sem_u))
        copies.append(pltpu.async_copy(
            fi.at[idx_v.at[dp]], vt.at[dp], sem_v))
    for c in copies:
        c.wait()

    wh_rows = [whv[pl.ds(d * 16, 16)] for d in range(D)]

    def chunk(c, carry):
        s = c * 16
        ucols, vcols = [], []
        for dp in range(D // 2):
            ua, ub = plsc.unpack(
                plsc.bitcast(ut[dp, pl.ds(s, 16)], jnp.bfloat16),
                format=plsc.PackFormat.INTERLEAVED)
            va, vb = plsc.unpack(
                plsc.bitcast(vt[dp, pl.ds(s, 16)], jnp.bfloat16),
                format=plsc.PackFormat.INTERLEAVED)
            ucols += [ua, ub]
            vcols += [va, vb]
        acc = jnp.zeros((16,), jnp.float32)
        for d in range(D):
            t = jnp.zeros((16,), jnp.float32)
            for e in range(D):
                t = t + wh_rows[d][e] * vcols[e]
            acc = acc + ucols[d] * t
        outv[pl.ds(s, 16)] = acc
        return carry

    lax.fori_loop(0, NCH, chunk, 0)
    pltpu.sync_copy(outv, out.at[pl.ds(base, BPW)])


def kernel(user_ids, item_ids, W_o, W_h, W_i):
    mesh = plsc.VectorSubcoreMesh(core_axis_name="c", subcore_axis_name="s")

    detile = pl.kernel(
        _detile_body,
        out_type=(jax.ShapeDtypeStruct((D // 2 * P,), jnp.int32),
                  jax.ShapeDtypeStruct((D // 2 * P,), jnp.int32)),
        mesh=mesh,
        compiler_params=pltpu.CompilerParams(needs_layout_passes=False),
        scratch_types=[
            pltpu.VMEM((8, PW), jnp.float32),
            pltpu.VMEM((8, PW), jnp.float32),
            pltpu.VMEM((4, PW), jnp.int32),
            pltpu.VMEM((4, PW), jnp.int32),
            pltpu.VMEM((D, 128), jnp.float32),
            pltpu.SemaphoreType.DMA,
            pltpu.SemaphoreType.DMA,
            pltpu.SemaphoreType.DMA,
            pltpu.SemaphoreType.DMA,
        ],
    )
    tails = jnp.stack([
        jnp.pad(W_o[NPF * PW:].T, ((0, 0), (0, 128 - TAIL))),
        jnp.pad(W_i[NPF * PW:].T, ((0, 0), (0, 128 - TAIL))),
    ])
    fo, fi = detile(W_o.T, W_i.T, tails)

    gather = pl.kernel(
        _gather_body,
        out_type=jax.ShapeDtypeStruct((B,), jnp.float32),
        mesh=mesh,
        compiler_params=pltpu.CompilerParams(
            needs_layout_passes=False, use_tc_tiling_on_sc=False),
        scratch_types=[
            pltpu.VMEM((BPW,), jnp.int32),
            pltpu.VMEM((BPW,), jnp.int32),
            pltpu.VMEM((D // 2, BPW), jnp.int32),
            pltpu.VMEM((D // 2, BPW), jnp.int32),
            pltpu.VMEM((D // 2, BPW), jnp.int32),
            pltpu.VMEM((D // 2, BPW), jnp.int32),
            pltpu.VMEM((D * D,), jnp.float32),
            pltpu.VMEM((BPW,), jnp.float32),
            pltpu.SemaphoreType.DMA,
            pltpu.SemaphoreType.DMA,
        ],
    )
    return gather(user_ids, item_ids, fo, W_h.reshape(D * D), fi)


# speedup vs baseline: 1.0244x; 1.0001x over previous
import jax
import jax.numpy as jnp
from jax import lax
from jax.experimental import pallas as pl
from jax.experimental.pallas import tpu as pltpu, tpu_sc as plsc

B = 16384
D = 16
NV = 1000000
P = 1000064
NC, NS = 2, 16
NW = NC * NS

PW = 3584
NPF = NV // PW
TAIL = NV - NPF * PW
NFULL = 4 * NPF
NI = (NFULL + NW - 1) // NW

BPW = B // NW
NCH = BPW // 16


def _detile_body(wot, wit, tails, fo, fi, stg0, stg1, outb0, outb1, tstg,
                 si0, si1, so0, so1):
    wid = lax.axis_index("s") * NC + lax.axis_index("c")
    stgs, outbs = [stg0, stg1], [outb0, outb1]
    sins, souts = [si0, si1], [so0, so1]

    def decode(u):
        return u % 2, (u // 2) % 2, u // 4

    def fire_in(u, par):
        tbl, g, piece = decode(u)
        c0 = piece * PW

        @pl.when(tbl == 0)
        def _():
            pltpu.async_copy(wot.at[pl.ds(g * 8, 8), pl.ds(c0, PW)],
                             stgs[par], sins[par])

        @pl.when(tbl == 1)
        def _():
            pltpu.async_copy(wit.at[pl.ds(g * 8, 8), pl.ds(c0, PW)],
                             stgs[par], sins[par])

    def wait_in(par):
        pltpu.make_async_copy(wot.at[pl.ds(0, 8), pl.ds(0, PW)],
                              stgs[par], sins[par]).wait()

    def extract(par):
        s, o = stgs[par], outbs[par]

        def row128(c128, carry):
            for h in range(8):
                sl = pl.ds(c128 * 128 + h * 16, 16)
                for p in range(4):
                    pk = plsc.pack(s[2 * p, sl], s[2 * p + 1, sl],
                                   format=plsc.PackFormat.INTERLEAVED)
                    o[p, sl] = plsc.bitcast(pk, jnp.int32)
            return carry

        lax.fori_loop(0, PW // 128, row128, 0)

    def fire_out(u, par):
        tbl, g, piece = decode(u)
        c0 = piece * PW
        for p in range(4):
            dst_off = (g * 4 + p) * P + c0

            @pl.when(tbl == 0)
            def _():
                pltpu.async_copy(outbs[par].at[p, pl.ds(0, PW)],
                                 fo.at[pl.ds(dst_off, PW)], souts[par])

            @pl.when(tbl == 1)
            def _():
                pltpu.async_copy(outbs[par].at[p, pl.ds(0, PW)],
                                 fi.at[pl.ds(dst_off, PW)], souts[par])

    def wait_out(par):
        for p in range(4):
            pltpu.make_async_copy(outbs[par].at[p, pl.ds(0, PW)],
                                  fo.at[pl.ds(0, PW)], souts[par]).wait()

    @pl.when(wid < NFULL)
    def _():
        fire_in(wid, 0)

    NI2 = (NI + 1) // 2

    def body(i2, carry):
        for par in range(2):
            i = 2 * i2 + par
            u = wid + NW * i
            un = wid + NW * (i + 1)

            @pl.when(un < NFULL)
            def _():
                fire_in(un, 1 - par)

            @pl.when((i >= 2) & (wid + NW * (i - 2) < NFULL))
            def _():
                wait_out(par)

            @pl.when(u < NFULL)
            def _():
                wait_in(par)
                extract(par)
                fire_out(u, par)

        return carry

    lax.fori_loop(0, NI2, body, 0)
    for j in (2 * NI2 - 2, 2 * NI2 - 1):
        @pl.when(wid + NW * j < NFULL)
        def _():
            wait_out(j % 2)

    def do_tail(tbl_idx, dst):
        pltpu.async_copy(tails.at[tbl_idx], tstg, si0).wait()

        def row16(c16, carry):
            sl = pl.ds(c16 * 16, 16)
            for dp in range(8):
                pk = plsc.pack(tstg[2 * dp, sl], tstg[2 * dp + 1, sl],
                               format=plsc.PackFormat.INTERLEAVED)
                outb0[dp % 4, pl.ds((dp // 4) * 128 + c16 * 16, 16)] = (
                    plsc.bitcast(pk, jnp.int32))
            return carry

        lax.fori_loop(0, TAIL // 16, row16, 0)
        cps = []
        for dp in range(8):
            cps.append(pltpu.async_copy(
                outb0.at[dp % 4, pl.ds((dp // 4) * 128, TAIL)],
                dst.at[pl.ds(dp * P + NPF * PW, TAIL)], so0))
        for c in cps:
            c.wait()

    @pl.when(wid == 0)
    def _():
        do_tail(0, fo)

    @pl.when(wid == 1)
    def _():
        do_tail(1, fi)


def _gather_body(uids, iids, fo, wh, fi, out, ids_u, ids_v, idx_u, idx_v,
                 ut, vt, whv, outv, sem_u, sem_v):
    wid = lax.axis_index("s") * NC + lax.axis_index("c")
    base = wid * BPW

    pltpu.sync_copy(uids.at[pl.ds(base, BPW)], ids_u)
    pltpu.sync_copy(iids.at[pl.ds(base, BPW)], ids_v)
    pltpu.sync_copy(wh, whv)

    copies = []
    for dp in range(D // 2):
        doff = jnp.full((16,), dp * P, jnp.int32)
        for t in range(BPW // 16):
            sl = pl.ds(t * 16, 16)
            idx_u[dp, sl] = ids_u[sl] + doff
            idx_v[dp, sl] = ids_v[sl] + doff
        copies.append(pltpu.async_copy(
            fo.at[idx_u.at[dp]], ut.at[dp], sem_u))
        copies.append(pltpu.async_copy(
            fi.at[idx_v.at[dp]], vt.at[dp], sem_v))
    for c in copies:
        c.wait()

    wh_rows = [whv[pl.ds(d * 16, 16)] for d in range(D)]

    def chunk(c, carry):
        s = c * 16
        ucols, vcols = [], []
        for dp in range(D // 2):
            ua, ub = plsc.unpack(
                plsc.bitcast(ut[dp, pl.ds(s, 16)], jnp.bfloat16),
                format=plsc.PackFormat.INTERLEAVED)
            va, vb = plsc.unpack(
                plsc.bitcast(vt[dp, pl.ds(s, 16)], jnp.bfloat16),
                format=plsc.PackFormat.INTERLEAVED)
            ucols += [ua, ub]
            vcols += [va, vb]
        acc = jnp.zeros((16,), jnp.float32)
        for d in range(D):
            t = jnp.zeros((16,), jnp.float32)
            for e in range(D):
                t = t + wh_rows[d][e] * vcols[e]
            acc = acc + ucols[d] * t
        outv[pl.ds(s, 16)] = acc
        return carry

    lax.fori_loop(0, NCH, chunk, 0)
    pltpu.sync_copy(outv, out.at[pl.ds(base, BPW)])


def kernel(user_ids, item_ids, W_o, W_h, W_i):
    mesh = plsc.VectorSubcoreMesh(core_axis_name="c", subcore_axis_name="s")

    detile = pl.kernel(
        _detile_body,
        out_type=(jax.ShapeDtypeStruct((D // 2 * P,), jnp.int32),
                  jax.ShapeDtypeStruct((D // 2 * P,), jnp.int32)),
        mesh=mesh,
        compiler_params=pltpu.CompilerParams(needs_layout_passes=False),
        scratch_types=[
            pltpu.VMEM((8, PW), jnp.float32),
            pltpu.VMEM((8, PW), jnp.float32),
            pltpu.VMEM((4, PW), jnp.int32),
            pltpu.VMEM((4, PW), jnp.int32),
            pltpu.VMEM((D, 128), jnp.float32),
            pltpu.SemaphoreType.DMA,
            pltpu.SemaphoreType.DMA,
            pltpu.SemaphoreType.DMA,
            pltpu.SemaphoreType.DMA,
        ],
    )
    tails = jnp.stack([
        jnp.pad(W_o[NPF * PW:].T, ((0, 0), (0, 128 - TAIL))),
        jnp.pad(W_i[NPF * PW:].T, ((0, 0), (0, 128 - TAIL))),
    ])
    fo, fi = detile(W_o.T, W_i.T, tails)

    gather = pl.kernel(
        _gather_body,
        out_type=jax.ShapeDtypeStruct((B,), jnp.float32),
        mesh=mesh,
        compiler_params=pltpu.CompilerParams(
            needs_layout_passes=False, use_tc_tiling_on_sc=False),
        scratch_types=[
            pltpu.VMEM((BPW,), jnp.int32),
            pltpu.VMEM((BPW,), jnp.int32),
            pltpu.VMEM((D // 2, BPW), jnp.int32),
            pltpu.VMEM((D // 2, BPW), jnp.int32),
            pltpu.VMEM((D // 2, BPW), jnp.int32),
            pltpu.VMEM((D // 2, BPW), jnp.int32),
            pltpu.VMEM((D * D,), jnp.float32),
            pltpu.VMEM((BPW,), jnp.float32),
            pltpu.SemaphoreType.DMA,
            pltpu.SemaphoreType.DMA,
        ],
    )
    return gather(user_ids, item_ids, fo, W_h.reshape(D * D), fi)
